# baseline (speedup 1.0000x reference)
import functools

import jax
import jax.numpy as jnp
from jax import lax
from jax.experimental import pallas as pl
from jax.experimental.pallas import tpu as pltpu
from jax.experimental.pallas import tpu_sc as plsc

NC = 2
NS = 16
NW = NC * NS
LANES = 16
CHUNK = 128


@functools.lru_cache(maxsize=None)
def _build_glove_sc(B: int, D: int):
    b_per_w = B // NW
    n_chunks = b_per_w // CHUNK
    n_seg = D // LANES
    mesh = plsc.VectorSubcoreMesh(
        core_axis_name="c", subcore_axis_name="s",
        num_cores=NC, num_subcores=NS,
    )

    @functools.partial(
        pl.kernel,
        out_type=jax.ShapeDtypeStruct((B,), jnp.float32),
        mesh=mesh,
        compiler_params=pltpu.CompilerParams(
            needs_layout_passes=False, use_tc_tiling_on_sc=False),
        scratch_types=[
            pltpu.VMEM((b_per_w,), jnp.int32),
            pltpu.VMEM((b_per_w,), jnp.int32),
            pltpu.VMEM((b_per_w, D), jnp.float32),
            pltpu.VMEM((b_per_w, D), jnp.float32),
            pltpu.VMEM((b_per_w,), jnp.float32),
            pltpu.VMEM((b_per_w,), jnp.float32),
            pltpu.VMEM((b_per_w,), jnp.float32),
            [pltpu.SemaphoreType.DMA] * 4,
        ],
    )
    def glove_kernel(tok_hbm, ctx_hbm, temb_hbm, cemb_hbm, tb_hbm, cb_hbm,
                     out_hbm, idx_i, idx_j, wi_v, wj_v, bi_v, bj_v,
                     out_v, sems):
        wid = lax.axis_index("s") * NC + lax.axis_index("c")
        base = wid * b_per_w

        pltpu.sync_copy(tok_hbm.at[pl.ds(base, b_per_w)], idx_i)
        pltpu.sync_copy(ctx_hbm.at[pl.ds(base, b_per_w)], idx_j)

        copies = []
        for c in range(n_chunks):
            sl = pl.ds(c * CHUNK, CHUNK)
            copies.append((
                pltpu.async_copy(temb_hbm.at[idx_i.at[sl]], wi_v.at[sl],
                                 sems[c]),
                pltpu.async_copy(cemb_hbm.at[idx_j.at[sl]], wj_v.at[sl],
                                 sems[c]),
                pltpu.async_copy(tb_hbm.at[idx_i.at[sl]], bi_v.at[sl],
                                 sems[c]),
                pltpu.async_copy(cb_hbm.at[idx_j.at[sl]], bj_v.at[sl],
                                 sems[c]),
            ))

        lane_ids = lax.iota(jnp.int32, LANES)

        def block(b, carry):
            r0 = b * LANES
            sums = jnp.zeros((LANES,), jnp.float32)
            for r in range(LANES):
                row = r0 + r
                acc = wi_v[row, pl.ds(0, LANES)] * wj_v[row, pl.ds(0, LANES)]
                for s in range(1, n_seg):
                    sl = pl.ds(s * LANES, LANES)
                    acc = acc + wi_v[row, sl] * wj_v[row, sl]
                sums = jnp.where(lane_ids == r, jnp.sum(acc), sums)
            blk = pl.ds(r0, LANES)
            out_v[blk] = sums + bi_v[blk] + bj_v[blk]
            return carry

        blocks_per_chunk = CHUNK // LANES
        for c in range(n_chunks):
            for cp in copies[c]:
                cp.wait()
            lax.fori_loop(c * blocks_per_chunk, (c + 1) * blocks_per_chunk,
                          block, 0)

        pltpu.sync_copy(out_v, out_hbm.at[pl.ds(base, b_per_w)])

    return glove_kernel


def kernel(token, context_token, token_embedding, context_embedding,
           token_bias, context_bias):
    B = token.shape[0]
    D = token_embedding.shape[1]
    tok = token.astype(jnp.int32)
    ctx = context_token.astype(jnp.int32)
    tb = token_bias.sum(axis=1)
    cb = context_bias.sum(axis=1)
    return _build_glove_sc(B, D)(tok, ctx, token_embedding,
                                 context_embedding, tb, cb)


# speedup vs baseline: 1.3204x; 1.3204x over previous
import functools

import jax
import jax.numpy as jnp
from jax import lax
from jax.experimental import pallas as pl
from jax.experimental.pallas import tpu as pltpu
from jax.experimental.pallas import tpu_sc as plsc

NC = 2
NS = 16
NW = NC * NS
LANES = 16
CHUNK = 128
N_PASS = 2


@functools.lru_cache(maxsize=None)
def _build_glove_sc(B: int, D: int):
    b_per_w = B // NW
    n_chunks = b_per_w // CHUNK
    n_seg = D // LANES
    rows_per_pass = b_per_w // N_PASS
    mesh = plsc.VectorSubcoreMesh(
        core_axis_name="c", subcore_axis_name="s",
        num_cores=NC, num_subcores=NS,
    )

    @functools.partial(
        pl.kernel,
        out_type=jax.ShapeDtypeStruct((B,), jnp.float32),
        mesh=mesh,
        compiler_params=pltpu.CompilerParams(
            needs_layout_passes=False, use_tc_tiling_on_sc=True),
        scratch_types=[
            pltpu.VMEM((b_per_w,), jnp.int32),
            pltpu.VMEM((b_per_w,), jnp.int32),
            pltpu.VMEM((rows_per_pass, D), jnp.float32),
            pltpu.VMEM((rows_per_pass, D), jnp.float32),
            pltpu.VMEM((b_per_w,), jnp.float32),
            pltpu.VMEM((b_per_w,), jnp.float32),
            pltpu.VMEM((b_per_w,), jnp.float32),
            [pltpu.SemaphoreType.DMA] * 4,
        ],
    )
    def glove_kernel(tok_hbm, ctx_hbm, temb_hbm, cemb_hbm, tb_hbm, cb_hbm,
                     out_hbm, idx_i, idx_j, wi_v, wj_v, bi_v, bj_v,
                     out_v, sems):
        wid = lax.axis_index("s") * NC + lax.axis_index("c")
        base = wid * b_per_w

        pltpu.sync_copy(tok_hbm.at[pl.ds(base, b_per_w)], idx_i)
        pltpu.sync_copy(ctx_hbm.at[pl.ds(base, b_per_w)], idx_j)

        for c in range(n_chunks):
            sl = pl.ds(c * CHUNK, CHUNK)
            pltpu.async_copy(tb_hbm.at[idx_i.at[sl]], bi_v.at[sl], sems[2])
            pltpu.async_copy(cb_hbm.at[idx_j.at[sl]], bj_v.at[sl], sems[3])

        lane_ids = lax.iota(jnp.int32, LANES)

        def issue(b, carry):
            r0 = b * LANES
            ti = idx_i[pl.ds(r0, LANES)]
            tj = idx_j[pl.ds(r0, LANES)]
            dr0 = (b % (rows_per_pass // LANES)) * LANES
            for r in range(LANES):
                dst = pl.ds(dr0 + r, 1)
                pltpu.async_copy(temb_hbm.at[pl.ds(ti[r], 1)],
                                 wi_v.at[dst], sems[0])
                pltpu.async_copy(cemb_hbm.at[pl.ds(tj[r], 1)],
                                 wj_v.at[dst], sems[1])
            return carry

        def block(b, carry):
            r0 = b * LANES
            dr0 = (b % (rows_per_pass // LANES)) * LANES
            sums = jnp.zeros((LANES,), jnp.float32)
            for r in range(LANES):
                row = dr0 + r
                acc = wi_v[row, pl.ds(0, LANES)] * wj_v[row, pl.ds(0, LANES)]
                for s in range(1, n_seg):
                    sl = pl.ds(s * LANES, LANES)
                    acc = acc + wi_v[row, sl] * wj_v[row, sl]
                sums = jnp.where(lane_ids == r, jnp.sum(acc), sums)
            blk = pl.ds(r0, LANES)
            out_v[blk] = sums + bi_v[blk] + bj_v[blk]
            return carry

        pltpu.make_async_copy(out_hbm.at[pl.ds(0, b_per_w)], bi_v,
                              sems[2]).wait()
        pltpu.make_async_copy(out_hbm.at[pl.ds(0, b_per_w)], bj_v,
                              sems[3]).wait()

        blocks_per_pass = rows_per_pass // LANES
        for p in range(N_PASS):
            lax.fori_loop(p * blocks_per_pass, (p + 1) * blocks_per_pass,
                          issue, 0)
            pltpu.make_async_copy(temb_hbm.at[pl.ds(0, rows_per_pass)],
                                  wi_v, sems[0]).wait()
            pltpu.make_async_copy(cemb_hbm.at[pl.ds(0, rows_per_pass)],
                                  wj_v, sems[1]).wait()
            lax.fori_loop(p * blocks_per_pass, (p + 1) * blocks_per_pass,
                          block, 0)

        pltpu.sync_copy(out_v, out_hbm.at[pl.ds(base, b_per_w)])

    return glove_kernel


def kernel(token, context_token, token_embedding, context_embedding,
           token_bias, context_bias):
    B = token.shape[0]
    D = token_embedding.shape[1]
    tok = token.astype(jnp.int32)
    ctx = context_token.astype(jnp.int32)
    tb = token_bias.sum(axis=1)
    cb = context_bias.sum(axis=1)
    return _build_glove_sc(B, D)(tok, ctx, token_embedding,
                                 context_embedding, tb, cb)


# speedup vs baseline: 1.3212x; 1.0006x over previous
import functools

import jax
import jax.numpy as jnp
from jax import lax
from jax.experimental import pallas as pl
from jax.experimental.pallas import tpu as pltpu
from jax.experimental.pallas import tpu_sc as plsc

NC = 2
NS = 16
NW = NC * NS
LANES = 16
CHUNK = 128
N_PASS = 2

_SC_PARAMS = dict(
    compiler_params=None,
)


def _mesh():
    return plsc.VectorSubcoreMesh(
        core_axis_name="c", subcore_axis_name="s",
        num_cores=NC, num_subcores=NS,
    )


def _cparams():
    return pltpu.CompilerParams(
        needs_layout_passes=False, use_tc_tiling_on_sc=True)


def _issue_rows(idx_v, table_hbm, dst_v, sem, b, rows_per_pass):
    r0 = b * LANES
    ti = idx_v[pl.ds(r0, LANES)]
    dr0 = (b % (rows_per_pass // LANES)) * LANES
    for r in range(LANES):
        pltpu.async_copy(table_hbm.at[pl.ds(ti[r], 1)],
                         dst_v.at[pl.ds(dr0 + r, 1)], sem)


@functools.lru_cache(maxsize=None)
def _build_k1(B: int, D: int):
    b_per_w = B // NW
    n_chunks = b_per_w // CHUNK
    rows_per_pass = b_per_w // N_PASS

    @functools.partial(
        pl.kernel,
        out_type=(jax.ShapeDtypeStruct((B, D), jnp.float32),
                  jax.ShapeDtypeStruct((B,), jnp.float32)),
        mesh=_mesh(),
        compiler_params=_cparams(),
        scratch_types=[
            pltpu.VMEM((b_per_w,), jnp.int32),
            pltpu.VMEM((rows_per_pass, D), jnp.float32),
            pltpu.VMEM((b_per_w,), jnp.float32),
            [pltpu.SemaphoreType.DMA] * 2,
        ],
    )
    def k1(tok_hbm, temb_hbm, tb_hbm, rows_out, bias_out,
           idx_i, wi_v, bi_v, sems):
        wid = lax.axis_index("s") * NC + lax.axis_index("c")
        base = wid * b_per_w
        pltpu.sync_copy(tok_hbm.at[pl.ds(base, b_per_w)], idx_i)
        for c in range(n_chunks):
            sl = pl.ds(c * CHUNK, CHUNK)
            pltpu.async_copy(tb_hbm.at[idx_i.at[sl]], bi_v.at[sl], sems[1])

        def issue(b, carry):
            _issue_rows(idx_i, temb_hbm, wi_v, sems[0], b, rows_per_pass)
            return carry

        blocks_per_pass = rows_per_pass // LANES
        for p in range(N_PASS):
            lax.fori_loop(p * blocks_per_pass, (p + 1) * blocks_per_pass,
                          issue, 0)
            pltpu.make_async_copy(temb_hbm.at[pl.ds(0, rows_per_pass)],
                                  wi_v, sems[0]).wait()
            pltpu.sync_copy(
                wi_v, rows_out.at[pl.ds(base + p * rows_per_pass,
                                        rows_per_pass)])
        pltpu.make_async_copy(tb_hbm.at[pl.ds(0, b_per_w)], bi_v,
                              sems[1]).wait()
        pltpu.sync_copy(bi_v, bias_out.at[pl.ds(base, b_per_w)])

    return k1


@functools.lru_cache(maxsize=None)
def _build_k2(B: int, D: int):
    b_per_w = B // NW
    n_chunks = b_per_w // CHUNK
    n_seg = D // LANES
    rows_per_pass = b_per_w // N_PASS

    @functools.partial(
        pl.kernel,
        out_type=jax.ShapeDtypeStruct((B,), jnp.float32),
        mesh=_mesh(),
        compiler_params=_cparams(),
        scratch_types=[
            pltpu.VMEM((b_per_w,), jnp.int32),
            pltpu.VMEM((rows_per_pass, D), jnp.float32),
            pltpu.VMEM((rows_per_pass, D), jnp.float32),
            pltpu.VMEM((b_per_w,), jnp.float32),
            pltpu.VMEM((b_per_w,), jnp.float32),
            pltpu.VMEM((b_per_w,), jnp.float32),
            [pltpu.SemaphoreType.DMA] * 3,
        ],
    )
    def k2(ctx_hbm, cemb_hbm, cb_hbm, wi_hbm, bi_hbm, out_hbm,
           idx_j, wi_v, wj_v, bi_v, bj_v, out_v, sems):
        wid = lax.axis_index("s") * NC + lax.axis_index("c")
        base = wid * b_per_w
        pltpu.sync_copy(ctx_hbm.at[pl.ds(base, b_per_w)], idx_j)
        for c in range(n_chunks):
            sl = pl.ds(c * CHUNK, CHUNK)
            pltpu.async_copy(cb_hbm.at[idx_j.at[sl]], bj_v.at[sl], sems[1])
        pltpu.sync_copy(bi_hbm.at[pl.ds(base, b_per_w)], bi_v)

        def issue(b, carry):
            _issue_rows(idx_j, cemb_hbm, wj_v, sems[0], b, rows_per_pass)
            return carry

        lane_ids = lax.iota(jnp.int32, LANES)

        def block(b, carry):
            r0 = b * LANES
            dr0 = (b % (rows_per_pass // LANES)) * LANES
            sums = jnp.zeros((LANES,), jnp.float32)
            for r in range(LANES):
                row = dr0 + r
                acc = wi_v[row, pl.ds(0, LANES)] * wj_v[row, pl.ds(0, LANES)]
                for s in range(1, n_seg):
                    sl = pl.ds(s * LANES, LANES)
                    acc = acc + wi_v[row, sl] * wj_v[row, sl]
                sums = jnp.where(lane_ids == r, jnp.sum(acc), sums)
            blk = pl.ds(r0, LANES)
            out_v[blk] = sums + bi_v[blk] + bj_v[blk]
            return carry

        pltpu.make_async_copy(cb_hbm.at[pl.ds(0, b_per_w)], bj_v,
                              sems[1]).wait()
        blocks_per_pass = rows_per_pass // LANES
        for p in range(N_PASS):
            lax.fori_loop(p * blocks_per_pass, (p + 1) * blocks_per_pass,
                          issue, 0)
            pltpu.async_copy(
                wi_hbm.at[pl.ds(base + p * rows_per_pass, rows_per_pass)],
                wi_v, sems[2])
            pltpu.make_async_copy(
                wi_hbm.at[pl.ds(0, rows_per_pass)], wi_v, sems[2]).wait()
            pltpu.make_async_copy(cemb_hbm.at[pl.ds(0, rows_per_pass)],
                                  wj_v, sems[0]).wait()
            lax.fori_loop(p * blocks_per_pass, (p + 1) * blocks_per_pass,
                          block, 0)
        pltpu.sync_copy(out_v, out_hbm.at[pl.ds(base, b_per_w)])

    return k2


def kernel(token, context_token, token_embedding, context_embedding,
           token_bias, context_bias):
    B = token.shape[0]
    D = token_embedding.shape[1]
    tok = token.astype(jnp.int32)
    ctx = context_token.astype(jnp.int32)
    tb = token_bias.sum(axis=1)
    cb = context_bias.sum(axis=1)
    wi_rows, bi = _build_k1(B, D)(tok, token_embedding, tb)
    return _build_k2(B, D)(ctx, context_embedding, cb, wi_rows, bi)
